# baseline (speedup 1.0000x reference)
import jax
import jax.numpy as jnp
from jax.experimental import pallas as pl
from jax.experimental.pallas import tpu as pltpu

H = 139
V = 53
L = 6
G = 256


def _gru_body(inp_ref, hidden_ref, emb_ref, wih_ref, whh_ref, bih_ref,
              bhh_ref, wdec_ref, bdec_ref, out_ref, hout_ref):
    idx = inp_ref[0]
    x = emb_ref[pl.ds(idx, 1), :]
    gh = []
    for l in range(L):
        g = jnp.dot(hidden_ref[l], whh_ref[l],
                    preferred_element_type=jnp.float32)
        gh.append(g + bhh_ref[l])
    for l in range(L):
        h = hidden_ref[l]
        gi = jnp.dot(x, wih_ref[l], preferred_element_type=jnp.float32)
        gi = gi + bih_ref[l]
        ghl = gh[l]
        s = jax.nn.sigmoid(gi + ghl)
        r = s[:, :G]
        z = s[:, G:2 * G]
        n = jnp.tanh(gi[:, 2 * G:] + r * ghl[:, 2 * G:])
        xw = (1.0 - z) * n + z * jnp.pad(h, ((0, 0), (0, G - H)))
        x = xw[:, :H]
        hout_ref[l] = x
    out = jnp.dot(x, wdec_ref[...], preferred_element_type=jnp.float32)
    out_ref[...] = out + bdec_ref[...]


def kernel(input, hidden, emb, W_ih, W_hh, b_ih, b_hh, W_dec, b_dec):
    def pack_w(w):
        w4 = w.reshape(L, 3, H, H).transpose(0, 3, 1, 2)
        w4 = jnp.pad(w4, ((0, 0), (0, 0), (0, 0), (0, G - H)))
        return w4.reshape(L, H, 3 * G)

    def pack_b(b):
        b3 = b.reshape(L, 3, H)
        b3 = jnp.pad(b3, ((0, 0), (0, 0), (0, G - H)))
        return b3.reshape(L, 1, 3 * G)

    wih_p = pack_w(W_ih)
    whh_p = pack_w(W_hh)
    bih_p = pack_b(b_ih)
    bhh_p = pack_b(b_hh)
    wdec_t = W_dec.T
    bdec = b_dec.reshape(1, V)
    idx = input.astype(jnp.int32)

    out, hout = pl.pallas_call(
        _gru_body,
        out_shape=[
            jax.ShapeDtypeStruct((1, V), jnp.float32),
            jax.ShapeDtypeStruct((L, 1, H), jnp.float32),
        ],
        in_specs=[
            pl.BlockSpec(memory_space=pltpu.SMEM),
            pl.BlockSpec(memory_space=pltpu.VMEM),
            pl.BlockSpec(memory_space=pltpu.VMEM),
            pl.BlockSpec(memory_space=pltpu.VMEM),
            pl.BlockSpec(memory_space=pltpu.VMEM),
            pl.BlockSpec(memory_space=pltpu.VMEM),
            pl.BlockSpec(memory_space=pltpu.VMEM),
            pl.BlockSpec(memory_space=pltpu.VMEM),
            pl.BlockSpec(memory_space=pltpu.VMEM),
        ],
        out_specs=[
            pl.BlockSpec(memory_space=pltpu.VMEM),
            pl.BlockSpec(memory_space=pltpu.VMEM),
        ],
    )(idx, hidden, emb, wih_p, whh_p, bih_p, bhh_p, wdec_t, bdec)
    return out, hout


# speedup vs baseline: 1.3759x; 1.3759x over previous
import jax
import jax.numpy as jnp
from jax.experimental import pallas as pl
from jax.experimental.pallas import tpu as pltpu

H = 139
V = 53
L = 6


def _gru_body(inp_ref, hidden_ref, emb_ref, wir_ref, wiz_ref, win_ref,
              whr_ref, whz_ref, whn_ref, bir_ref, biz_ref, bin_ref,
              bhr_ref, bhz_ref, bhn_ref, wdec_ref, bdec_ref,
              out_ref, hout_ref):
    idx = inp_ref[0]
    x = emb_ref[pl.ds(idx, 1), :]
    ghr, ghz, ghn = [], [], []
    for l in range(L):
        h = hidden_ref[l]
        ghr.append(jnp.dot(h, whr_ref[l], preferred_element_type=jnp.float32)
                   + bhr_ref[l])
        ghz.append(jnp.dot(h, whz_ref[l], preferred_element_type=jnp.float32)
                   + bhz_ref[l])
        ghn.append(jnp.dot(h, whn_ref[l], preferred_element_type=jnp.float32)
                   + bhn_ref[l])
    for l in range(L):
        h = hidden_ref[l]
        gir = jnp.dot(x, wir_ref[l], preferred_element_type=jnp.float32)
        giz = jnp.dot(x, wiz_ref[l], preferred_element_type=jnp.float32)
        gin = jnp.dot(x, win_ref[l], preferred_element_type=jnp.float32)
        r = jax.nn.sigmoid(gir + bir_ref[l] + ghr[l])
        z = jax.nn.sigmoid(giz + biz_ref[l] + ghz[l])
        n = jnp.tanh(gin + bin_ref[l] + r * ghn[l])
        x = (1.0 - z) * n + z * h
        hout_ref[l] = x
    out = jnp.dot(x, wdec_ref[...], preferred_element_type=jnp.float32)
    out_ref[...] = out + bdec_ref[...]


def kernel(input, hidden, emb, W_ih, W_hh, b_ih, b_hh, W_dec, b_dec):
    wir = W_ih[:, :H, :].transpose(0, 2, 1)
    wiz = W_ih[:, H:2 * H, :].transpose(0, 2, 1)
    win = W_ih[:, 2 * H:, :].transpose(0, 2, 1)
    whr = W_hh[:, :H, :].transpose(0, 2, 1)
    whz = W_hh[:, H:2 * H, :].transpose(0, 2, 1)
    whn = W_hh[:, 2 * H:, :].transpose(0, 2, 1)
    bir = b_ih[:, :H].reshape(L, 1, H)
    biz = b_ih[:, H:2 * H].reshape(L, 1, H)
    bin_ = b_ih[:, 2 * H:].reshape(L, 1, H)
    bhr = b_hh[:, :H].reshape(L, 1, H)
    bhz = b_hh[:, H:2 * H].reshape(L, 1, H)
    bhn = b_hh[:, 2 * H:].reshape(L, 1, H)
    wdec_t = W_dec.T
    bdec = b_dec.reshape(1, V)
    idx = input.astype(jnp.int32)

    vm = pl.BlockSpec(memory_space=pltpu.VMEM)
    out, hout = pl.pallas_call(
        _gru_body,
        out_shape=[
            jax.ShapeDtypeStruct((1, V), jnp.float32),
            jax.ShapeDtypeStruct((L, 1, H), jnp.float32),
        ],
        in_specs=[pl.BlockSpec(memory_space=pltpu.SMEM)] + [vm] * 16,
        out_specs=[vm, vm],
    )(idx, hidden, emb, wir, wiz, win, whr, whz, whn,
      bir, biz, bin_, bhr, bhz, bhn, wdec_t, bdec)
    return out, hout


# speedup vs baseline: 2.4244x; 1.7620x over previous
import jax
import jax.numpy as jnp
from jax.experimental import pallas as pl
from jax.experimental.pallas import tpu as pltpu

H = 139
V = 53
L = 6


def _gru_body(inp_ref, hidden_ref, emb_ref, wih_ref, whh_ref, bih_ref,
              bhh_ref, wdec_ref, bdec_ref, out_ref, hout_ref):
    idx = inp_ref[0]
    x = emb_ref[pl.ds(idx, 1), :]
    wih_b = [wih_ref[l].astype(jnp.bfloat16) for l in range(L)]
    whh_b = [whh_ref[l].astype(jnp.bfloat16) for l in range(L)]
    gh = []
    for l in range(L):
        g = jnp.dot(hidden_ref[l].astype(jnp.bfloat16), whh_b[l],
                    preferred_element_type=jnp.float32)
        gh.append(g + bhh_ref[l])
    for l in range(L):
        h = hidden_ref[l]
        gi = jnp.dot(x.astype(jnp.bfloat16), wih_b[l],
                     preferred_element_type=jnp.float32)
        gi = gi + bih_ref[l]
        ghl = gh[l]
        r = jax.nn.sigmoid(gi[:, :H] + ghl[:, :H])
        z = jax.nn.sigmoid(gi[:, H:2 * H] + ghl[:, H:2 * H])
        n = jnp.tanh(gi[:, 2 * H:] + r * ghl[:, 2 * H:])
        x = (1.0 - z) * n + z * h
        hout_ref[l] = x
    out = jnp.dot(x.astype(jnp.bfloat16), wdec_ref[...].astype(jnp.bfloat16),
                  preferred_element_type=jnp.float32)
    out_ref[...] = out + bdec_ref[...]


def kernel(input, hidden, emb, W_ih, W_hh, b_ih, b_hh, W_dec, b_dec):
    wih_t = W_ih.transpose(0, 2, 1)
    whh_t = W_hh.transpose(0, 2, 1)
    bih = b_ih.reshape(L, 1, 3 * H)
    bhh = b_hh.reshape(L, 1, 3 * H)
    wdec_t = W_dec.T
    bdec = b_dec.reshape(1, V)
    idx = input.astype(jnp.int32)

    out, hout = pl.pallas_call(
        _gru_body,
        out_shape=[
            jax.ShapeDtypeStruct((1, V), jnp.float32),
            jax.ShapeDtypeStruct((L, 1, H), jnp.float32),
        ],
        in_specs=[
            pl.BlockSpec(memory_space=pltpu.SMEM),
            pl.BlockSpec(memory_space=pltpu.VMEM),
            pl.BlockSpec(memory_space=pltpu.VMEM),
            pl.BlockSpec(memory_space=pltpu.VMEM),
            pl.BlockSpec(memory_space=pltpu.VMEM),
            pl.BlockSpec(memory_space=pltpu.VMEM),
            pl.BlockSpec(memory_space=pltpu.VMEM),
            pl.BlockSpec(memory_space=pltpu.VMEM),
            pl.BlockSpec(memory_space=pltpu.VMEM),
        ],
        out_specs=[
            pl.BlockSpec(memory_space=pltpu.VMEM),
            pl.BlockSpec(memory_space=pltpu.VMEM),
        ],
    )(idx, hidden, emb, wih_t, whh_t, bih, bhh, wdec_t, bdec)
    return out, hout
